# baseline (speedup 1.0000x reference)
import functools

import jax
import jax.numpy as jnp
from jax import lax
from jax.experimental import pallas as pl
from jax.experimental.pallas import tpu as pltpu
from jax.experimental.pallas import tpu_sc as plsc

D = 768
H = 3072
N = 8192
T = 512
_KS = (128, 256, 384, 768)

_info = plsc.get_sparse_core_info()
_NC, _NS = _info.num_cores, _info.num_subcores
_NW = _NC * _NS
_RPW = N // _NW
_CH = 128


def _routing_body(tm_ref, pos_ref, es_ref):
    tm = tm_ref[...]
    R, C = tm.shape
    iu = lax.broadcasted_iota(jnp.int32, (C, C), 0)
    ju = lax.broadcasted_iota(jnp.int32, (C, C), 1)
    tri_u = (iu <= ju).astype(jnp.float32)
    il = lax.broadcasted_iota(jnp.int32, (R, R), 0)
    jl = lax.broadcasted_iota(jnp.int32, (R, R), 1)
    tri_l = (jl <= il).astype(jnp.float32)
    pos = jnp.zeros((R, C), jnp.float32)
    offset = jnp.float32(0.0)
    ccs = []
    for m in range(4):
        a = (tm == m).astype(jnp.float32)
        rc = jnp.dot(a, tri_u, preferred_element_type=jnp.float32)
        rt = rc[:, C - 1:C]
        co = jnp.dot(tri_l, rt, preferred_element_type=jnp.float32)
        rank = rc - a + (co - rt)
        tot = jnp.sum(a)
        pos = pos + a * (offset + rank)
        offset = offset + tot
        ccs.append(offset)
    pos_ref[...] = pos.astype(jnp.int32)
    p = (lax.broadcasted_iota(jnp.int32, (R, C), 0) * C +
         lax.broadcasted_iota(jnp.int32, (R, C), 1)).astype(jnp.float32)
    es = jnp.zeros((R, C), jnp.int32)
    for m in range(4):
        es = es + (p >= ccs[m]).astype(jnp.int32)
    es_ref[...] = es


_sc_mesh = plsc.VectorSubcoreMesh(core_axis_name="c", subcore_axis_name="s")


@functools.partial(
    pl.kernel, mesh=_sc_mesh,
    out_type=jax.ShapeDtypeStruct((N, D), jnp.float32),
    scratch_types=[
        pltpu.VMEM((_CH,), jnp.int32),
        pltpu.VMEM((_CH, D), jnp.float32),
        pltpu.SemaphoreType.DMA,
    ],
)
def _dispatch(x_hbm, pos_hbm, xs_hbm, idx_v, rows_v, sem):
    wid = lax.axis_index("s") * _NC + lax.axis_index("c")
    for j in range(_RPW // _CH):
        base = wid * _RPW + j * _CH
        pltpu.sync_copy(pos_hbm.at[pl.ds(base, _CH)], idx_v)
        pltpu.sync_copy(x_hbm.at[pl.ds(base, _CH)], rows_v)
        pltpu.async_copy(rows_v, xs_hbm.at[idx_v], sem).wait()


@functools.partial(
    pl.kernel, mesh=_sc_mesh,
    out_type=jax.ShapeDtypeStruct((N, D), jnp.float32),
    scratch_types=[
        pltpu.VMEM((_CH,), jnp.int32),
        pltpu.VMEM((_CH, D), jnp.float32),
        pltpu.SemaphoreType.DMA,
    ],
)
def _combine(ys_hbm, pos_hbm, out_hbm, idx_v, rows_v, sem):
    wid = lax.axis_index("s") * _NC + lax.axis_index("c")
    for j in range(_RPW // _CH):
        base = wid * _RPW + j * _CH
        pltpu.sync_copy(pos_hbm.at[pl.ds(base, _CH)], idx_v)
        pltpu.async_copy(ys_hbm.at[idx_v], rows_v, sem).wait()
        pltpu.sync_copy(rows_v, out_hbm.at[pl.ds(base, _CH)])


def _ffn_body(x_ref, es_ref, w1_ref, b1_ref, w2_ref, b2_ref, out_ref,
              w1t_ref, w2t_ref):
    Tb, Dd = x_ref.shape

    @pl.when(pl.program_id(0) == 0)
    def _():
        w1t_ref[...] = w1_ref[...].astype(jnp.bfloat16).T
        w2t_ref[...] = w2_ref[...].astype(jnp.bfloat16).T

    es = es_ref[...]
    be = es_ref[Tb - 1, 0]
    thresh = jnp.where(es == 0, 96,
             jnp.where(es == 1, 192,
             jnp.where(es == 2, 384, 768)))
    b1 = b1_ref[...]
    for m in range(4):
        K = _KS[m]
        @pl.when(be == m)
        def _(K=K):
            col = lax.broadcasted_iota(jnp.int32, (Tb, K), 1)
            mask = col < thresh
            xm = jnp.where(mask, x_ref[:, :K], 0.0).astype(jnp.bfloat16)
            h = jnp.dot(xm, w1t_ref[:K, :], preferred_element_type=jnp.float32)
            h = h + b1
            h = 0.5 * h * (1.0 + lax.erf(h * 0.7071067811865476))
            y = jnp.dot(h.astype(jnp.bfloat16), w2t_ref[:, :K],
                        preferred_element_type=jnp.float32)
            y = y + b2_ref[:, :K]
            out_ref[:, :K] = jnp.where(mask, y, 0.0)
            if K < Dd:
                out_ref[:, K:] = jnp.zeros((Tb, Dd - K), jnp.float32)


@functools.partial(jax.jit, static_argnames=())
def kernel(x, token_mask, w1, b1, w2, b2):
    B, S, Dd = x.shape
    xf = x.reshape(N, D)
    tm2d = token_mask.reshape(64, 128).astype(jnp.int32)
    pos, es = pl.pallas_call(
        _routing_body,
        out_shape=(jax.ShapeDtypeStruct((64, 128), jnp.int32),
                   jax.ShapeDtypeStruct((64, 128), jnp.int32)),
    )(tm2d)
    posf = pos.reshape(N)
    xs = _dispatch(xf, posf)
    b1r = b1.reshape(1, H)
    b2r = b2.reshape(1, D)
    es2 = es.reshape(N, 1)
    ys = pl.pallas_call(
        _ffn_body,
        grid=(N // T,),
        in_specs=[
            pl.BlockSpec((T, D), lambda i: (i, 0)),
            pl.BlockSpec((T, 1), lambda i: (i, 0)),
            pl.BlockSpec((H, D), lambda i: (0, 0)),
            pl.BlockSpec((1, H), lambda i: (0, 0)),
            pl.BlockSpec((D, H), lambda i: (0, 0)),
            pl.BlockSpec((1, D), lambda i: (0, 0)),
        ],
        out_specs=pl.BlockSpec((T, D), lambda i: (i, 0)),
        out_shape=jax.ShapeDtypeStruct((N, D), jnp.float32),
        scratch_shapes=[
            pltpu.VMEM((D, H), jnp.bfloat16),
            pltpu.VMEM((H, D), jnp.bfloat16),
        ],
        compiler_params=pltpu.CompilerParams(
            dimension_semantics=("arbitrary",),
        ),
    )(xs, es2, w1, b1r, w2, b2r)
    out = _combine(ys, posf)
    return out.reshape(B, S, Dd)
